# Initial kernel scaffold; baseline (speedup 1.0000x reference)
import jax
import jax.numpy as jnp
from jax.experimental import pallas as pl


def kernel(pred, heatmaps, depth, loss_1, pred_depth, epoch):
    raise NotImplementedError("write your pallas kernel here")


# speedup vs baseline: 494.8972x; 494.8972x over previous
import functools

import jax
import jax.numpy as jnp
from jax import lax
from jax.experimental import pallas as pl
from jax.experimental.pallas import tpu as pltpu
from jax.experimental.pallas import tpu_sc as plsc

_IGNORE = 255
_NCLS = 19
_M = 8
_D = 10
_K = 26
_NSEG = _M * _M * _K
_SEGP = 1792
_TBL_ONE = _NCLS * _SEGP
_TBL = 3 * _TBL_ONE
_B, _W, _H = 4, 512, 512
_N = _B * _W * _H
_NW = 32
_CHUNK = _N // _NW
_SUB = 2048
_NSUB = _CHUNK // _SUB
_MG = 14592
_MROUNDS = _TBL // _MG
_SLICE = _MG // 16


def _prep_body(pred_ref, hm_ref, depth_ref, loss_ref, pd_ref, packed_ref, sums_ref):
    i = pl.program_id(0)
    j = pl.program_id(1)

    rows = lax.broadcasted_iota(jnp.int32, (256, 512), 0) + j * 256
    cols = lax.broadcasted_iota(jnp.int32, (256, 512), 1)
    rb = rows // 64
    cb = cols // 64
    row_in = (rows - rb * 64) < 63
    col_in = (cols - cb * 64) < 63
    blk = rb * _M + cb

    d = depth_ref[0]
    hm = hm_ref[0]
    pr = pred_ref[0]
    l1 = loss_ref[0]
    pd = pd_ref[0]

    kb = jnp.clip(jnp.floor(d / float(_D)).astype(jnp.int32), 0, _K - 1)
    kbf = kb.astype(jnp.float32)
    bin_valid = (d > kbf * _D) & (d < kbf * _D + (_D - 1))
    valid = hm != _IGNORE
    overall = row_in & col_in & valid & bin_valid
    seg = jnp.where(overall, blk * _K + kb, _NSEG)

    hmc = jnp.clip(hm, 0, _NCLS - 1)
    prc = jnp.clip(pr, 0, _NCLS - 1)
    il = hmc * _SEGP + seg
    ip = prc * _SEGP + seg
    packed_ref[0] = il | (ip << 16)

    diff = jnp.log(pd * 255.0 + 1.0) - jnp.log(d + 1.0)
    s0 = jnp.sum(diff * diff)
    s1 = jnp.sum(diff)
    s2 = jnp.sum(l1 * jnp.abs(diff))
    s3 = jnp.sum(valid.astype(jnp.float32))

    @pl.when((i == 0) & (j == 0))
    def _():
        sums_ref[0] = 0.0
        sums_ref[1] = 0.0
        sums_ref[2] = 0.0
        sums_ref[3] = 0.0

    sums_ref[0] += s0
    sums_ref[1] += s1
    sums_ref[2] += s2
    sums_ref[3] += s3


_prep = pl.pallas_call(
    _prep_body,
    grid=(_B, 2),
    in_specs=[
        pl.BlockSpec((1, 256, 512), lambda i, j: (i, j, 0)),
        pl.BlockSpec((1, 256, 512), lambda i, j: (i, j, 0)),
        pl.BlockSpec((1, 256, 512), lambda i, j: (i, j, 0)),
        pl.BlockSpec((1, 256, 512), lambda i, j: (i, j, 0)),
        pl.BlockSpec((1, 256, 512), lambda i, j: (i, j, 0)),
    ],
    out_specs=[
        pl.BlockSpec((1, 256, 512), lambda i, j: (i, j, 0)),
        pl.BlockSpec(memory_space=pltpu.SMEM, block_shape=(4,), index_map=lambda i, j: (0,)),
    ],
    out_shape=[
        jax.ShapeDtypeStruct((_B, _W, _H), jnp.int32),
        jax.ShapeDtypeStruct((4,), jnp.float32),
    ],
)


def _sc_body(packed_hbm, loss_hbm, out_hbm, tbl, pk0, pk1, ls0, ls1, acc, sbuf, stage, sem0, sem1, sem2):
    core = lax.axis_index("c")
    sid = lax.axis_index("s")
    wid = core * 16 + sid
    base = wid * _CHUNK

    zf = jnp.zeros((16,), jnp.float32)
    onef = jnp.ones((16,), jnp.float32)
    mask16 = jnp.full((16,), 0xFFFF, jnp.int32)
    shift16 = jnp.full((16,), 16, jnp.int32)
    off_p = jnp.full((16,), _TBL_ONE, jnp.int32)
    off_l = jnp.full((16,), 2 * _TBL_ONE, jnp.int32)

    def zbody(i, c):
        tbl[pl.ds(i * 64, 16)] = zf
        tbl[pl.ds(i * 64 + 16, 16)] = zf
        tbl[pl.ds(i * 64 + 32, 16)] = zf
        tbl[pl.ds(i * 64 + 48, 16)] = zf
        return c

    lax.fori_loop(0, _TBL // 64, zbody, 0, unroll=4)

    bufs = ((pk0, ls0, sem0), (pk1, ls1, sem1))

    def start(k, b):
        pk, ls, sem = bufs[b]
        h0 = pltpu.async_copy(packed_hbm.at[pl.ds(base + k * _SUB, _SUB)], pk, sem)
        h1 = pltpu.async_copy(loss_hbm.at[pl.ds(base + k * _SUB, _SUB)], ls, sem)
        return (h0, h1)

    def process(b):
        pk, ls, _ = bufs[b]

        def pbody(i, c):
            v = pk[pl.ds(i * 16, 16)]
            il = v & mask16
            ip = lax.shift_right_logical(v, shift16)
            lv = ls[pl.ds(i * 16, 16)]
            plsc.addupdate_scatter(tbl, [il], onef)
            plsc.addupdate_scatter(tbl, [ip + off_p], onef)
            plsc.addupdate_scatter(tbl, [il + off_l], lv)
            return c

        lax.fori_loop(0, _SUB // 16, pbody, 0, unroll=2)

    pending = start(0, 0)
    for k in range(_NSUB):
        b = k & 1
        cur = pending
        if k + 1 < _NSUB:
            pending = start(k + 1, 1 - b)
        cur[0].wait()
        cur[1].wait()
        process(b)

    for r in range(_MROUNDS):
        pltpu.sync_copy(
            tbl.at[pl.ds(r * _MG, _MG)],
            stage.at[pl.ds(sid * _MG, _MG)],
        )
        plsc.subcore_barrier()

        pltpu.sync_copy(stage.at[pl.ds(sid * _SLICE, _SLICE)], acc)
        for t in range(1, 16):
            pltpu.sync_copy(stage.at[pl.ds(t * _MG + sid * _SLICE, _SLICE)], sbuf)

            def rbody(i, c):
                acc[pl.ds(i * 16, 16)] += sbuf[pl.ds(i * 16, 16)]
                return c

            lax.fori_loop(0, _SLICE // 16, rbody, 0, unroll=4)

        pltpu.sync_copy(
            acc, out_hbm.at[pl.ds(core * _TBL + r * _MG + sid * _SLICE, _SLICE)]
        )
        plsc.subcore_barrier()


_sc_scatter = functools.partial(
    pl.kernel,
    out_type=jax.ShapeDtypeStruct((2 * _TBL,), jnp.float32),
    mesh=plsc.VectorSubcoreMesh(
        core_axis_name="c", subcore_axis_name="s", num_cores=2, num_subcores=16
    ),
    scratch_types=[
        pltpu.VMEM((_TBL,), jnp.float32),
        pltpu.VMEM((_SUB,), jnp.int32),
        pltpu.VMEM((_SUB,), jnp.int32),
        pltpu.VMEM((_SUB,), jnp.float32),
        pltpu.VMEM((_SUB,), jnp.float32),
        pltpu.VMEM((_SLICE,), jnp.float32),
        pltpu.VMEM((_SLICE,), jnp.float32),
        pltpu.VMEM_SHARED((16 * _MG,), jnp.float32),
        pltpu.SemaphoreType.DMA,
        pltpu.SemaphoreType.DMA,
        pltpu.SemaphoreType.DMA,
    ],
    compiler_params=pltpu.CompilerParams(needs_layout_passes=False),
)(_sc_body)


def _finish_body(sums_ref, tab_ref, out_ref):
    t = tab_ref[0] + tab_ref[1]
    cl = t[0:_NCLS, :]
    cp = t[_NCLS : 2 * _NCLS, :]
    lt = t[2 * _NCLS : 3 * _NCLS, :]

    counts = jnp.sum(cl, axis=0, keepdims=True)
    lsum = jnp.sum(lt, axis=0, keepdims=True)
    sym = jnp.sum(((cl > 0.0) != (cp > 0.0)).astype(jnp.float32), axis=0, keepdims=True)

    colid = lax.broadcasted_iota(jnp.int32, (1, _SEGP), 1)
    segvalid = colid < _NSEG
    active = (counts > 0.0) & segvalid
    mean = lsum / jnp.maximum(counts, 1.0)
    times = jnp.sum(active.astype(jnp.float32))
    loss3 = jnp.sum(jnp.where(active, sym * mean, 0.0)) / (times + 0.001)

    nf = float(_N)
    data_loss = sums_ref[0] / nf - 0.5 * (sums_ref[1] * sums_ref[1]) / (nf * nf)
    lr = sums_ref[2] / sums_ref[3]
    out_ref[0] = data_loss
    out_ref[1] = lr
    out_ref[2] = loss3


_finish = pl.pallas_call(
    _finish_body,
    in_specs=[
        pl.BlockSpec(memory_space=pltpu.SMEM),
        pl.BlockSpec((2, 3 * _NCLS, _SEGP), lambda: (0, 0, 0)),
    ],
    out_specs=pl.BlockSpec(memory_space=pltpu.SMEM),
    out_shape=jax.ShapeDtypeStruct((3,), jnp.float32),
)


def _run(pred, heatmaps, depth, loss_1, pred_depth):
    pred = pred.astype(jnp.int32)
    heatmaps = heatmaps.astype(jnp.int32)
    loss32 = loss_1.astype(jnp.float32)
    packed, sums = _prep(
        pred, heatmaps, depth.astype(jnp.float32), loss32, pred_depth.astype(jnp.float32)
    )
    tabs = _sc_scatter(packed.reshape(_N), loss32.reshape(_N))
    return _finish(sums, tabs.reshape(2, 3 * _NCLS, _SEGP))


def kernel(pred, heatmaps, depth, loss_1, pred_depth, epoch):
    odt = jnp.result_type(depth.dtype, jnp.float32)
    with jax.enable_x64(False):
        out = _run(pred, heatmaps, depth, loss_1, pred_depth)
    out = out.astype(odt)
    return out[0], out[1], out[2]
